# baseline (speedup 1.0000x reference)
import functools

import jax
import jax.numpy as jnp
import numpy as np
from jax import lax
from jax.experimental import pallas as pl
from jax.experimental.pallas import tpu as pltpu
from jax.experimental.pallas import tpu_sc as plsc

_N = 10000
_NPAD = 10240
_ROWS = _NPAD // 128
_IN_C = 128
_OUT_C = 128
_M = 2500
_MPAD = 2560
_K = 16
_QB = 256


def _fps_body(px_ref, py_ref, pz_ref, idx_ref, sx_ref, sy_ref, sz_ref):
    iota = jax.lax.broadcasted_iota(jnp.int32, (_ROWS, 128), 0) * 128 + \
           jax.lax.broadcasted_iota(jnp.int32, (_ROWS, 128), 1)
    iota_f = iota.astype(jnp.float32)
    valid = iota < _N
    BIGF = jnp.float32(1 << 30)
    ones_col = jnp.ones((128, 1), jnp.float32)
    dn = (((1,), (0,)), ((), ()))

    px0 = px_ref[0, 0]
    py0 = py_ref[0, 0]
    pz0 = pz_ref[0, 0]
    dx = px_ref[...] - px0
    dy = py_ref[...] - py0
    dz = pz_ref[...] - pz0
    d = jnp.where(valid, dx * dx + dy * dy + dz * dz, -jnp.inf)
    idx_ref[0] = 0
    sx_ref[0] = px0
    sy_ref[0] = py0
    sz_ref[0] = pz0

    def fold10(t):
        acc = t[0:8]
        for g in range(1, 10):
            acc = acc + t[8 * g:8 * g + 8]
        return acc

    def body(i, d):
        maxd = jnp.max(d)
        maskb = d == maxd
        m5 = jnp.concatenate([
            fold10(jnp.where(maskb, px_ref[...], 0.0)),
            fold10(jnp.where(maskb, py_ref[...], 0.0)),
            fold10(jnp.where(maskb, pz_ref[...], 0.0)),
            fold10(jnp.where(maskb, 1.0, 0.0)),
            fold10(jnp.where(maskb, iota_f, 0.0)),
        ], axis=0)
        s = jax.lax.dot_general(m5, ones_col, dn,
                                precision=jax.lax.Precision.HIGHEST,
                                preferred_element_type=jnp.float32)
        pxs = jnp.sum(s[0:8])
        pys = jnp.sum(s[8:16])
        pzs = jnp.sum(s[16:24])
        cnt = jnp.sum(s[24:32])
        nxs = jnp.sum(s[32:40])

        def fast(_):
            return nxs, pxs, pys, pzs

        def slow(_):
            nxtf = jnp.min(jnp.where(maskb, iota_f, BIGF))
            m2 = iota_f == nxtf
            m3 = jnp.concatenate([
                fold10(jnp.where(m2, px_ref[...], 0.0)),
                fold10(jnp.where(m2, py_ref[...], 0.0)),
                fold10(jnp.where(m2, pz_ref[...], 0.0)),
            ], axis=0)
            s2 = jax.lax.dot_general(m3, ones_col, dn,
                                     precision=jax.lax.Precision.HIGHEST,
                                     preferred_element_type=jnp.float32)
            return (nxtf, jnp.sum(s2[0:8]), jnp.sum(s2[8:16]),
                    jnp.sum(s2[16:24]))

        nxtf, pxv, pyv, pzv = jax.lax.cond(cnt > 1.5, slow, fast, None)
        nxt = nxtf.astype(jnp.int32)
        idx_ref[i] = nxt
        sx_ref[i] = pxv
        sy_ref[i] = pyv
        sz_ref[i] = pzv
        dx = px_ref[...] - pxv
        dy = py_ref[...] - pyv
        dz = pz_ref[...] - pzv
        return jnp.minimum(d, dx * dx + dy * dy + dz * dz)

    jax.lax.fori_loop(1, _M, body, d)


def _fps(px, py, pz):
    return pl.pallas_call(
        _fps_body,
        in_specs=[pl.BlockSpec(memory_space=pltpu.VMEM)] * 3,
        out_specs=[pl.BlockSpec(memory_space=pltpu.SMEM)] * 4,
        out_shape=[
            jax.ShapeDtypeStruct((_M,), jnp.int32),
            jax.ShapeDtypeStruct((_M,), jnp.float32),
            jax.ShapeDtypeStruct((_M,), jnp.float32),
            jax.ShapeDtypeStruct((_M,), jnp.float32),
        ],
    )(px, py, pz)


def _knn_body(q_ref, p_ref, out_ref, d2_ref):
    col_iota = jax.lax.broadcasted_iota(jnp.int32, (_QB, _NPAD), 1)
    lane16 = jax.lax.broadcasted_iota(jnp.int32, (_QB, _K), 1)
    BIG = jnp.int32(1 << 30)
    INF = jnp.float32(jnp.inf)

    ptx = p_ref[0:1, :]
    pty = p_ref[1:2, :]
    ptz = p_ref[2:3, :]
    pnorm = ptx * ptx + pty * pty + ptz * ptz
    pn_iota = jax.lax.broadcasted_iota(jnp.int32, (1, _NPAD), 1)
    pnorm = jnp.where(pn_iota < _N, pnorm, INF)

    q = q_ref[...]
    qx = q[:, 0:1]
    qy = q[:, 1:2]
    qz = q[:, 2:3]
    qnorm = qx * qx + qy * qy + qz * qz

    cross = jnp.dot(q, p_ref[...], preferred_element_type=jnp.float32)
    d2_ref[...] = (qnorm + pnorm) - 2.0 * cross

    def body(k, carry):
        acc, prev = carry
        d2 = d2_ref[...]
        d2 = jnp.where(col_iota == prev, INF, d2)
        d2_ref[...] = d2
        mn = jnp.min(d2, axis=1, keepdims=True)
        idx = jnp.min(jnp.where(d2 == mn, col_iota, BIG), axis=1, keepdims=True)
        acc = jnp.where(lane16 == k, idx, acc)
        return acc, idx

    acc0 = jnp.zeros((_QB, _K), jnp.int32)
    prev0 = jnp.full((_QB, 1), -1, jnp.int32)
    acc, _ = jax.lax.fori_loop(0, _K, body, (acc0, prev0))
    out_ref[...] = acc


def _knn(spos_pad, p3t):
    return pl.pallas_call(
        _knn_body,
        grid=(_MPAD // _QB,),
        in_specs=[
            pl.BlockSpec((_QB, 3), lambda i: (i, 0)),
            pl.BlockSpec((3, _NPAD), lambda i: (0, 0)),
        ],
        out_specs=pl.BlockSpec((_QB, _K), lambda i: (i, 0)),
        out_shape=jax.ShapeDtypeStruct((_MPAD, _K), jnp.int32),
        scratch_shapes=[pltpu.VMEM((_QB, _NPAD), jnp.float32)],
    )(spos_pad, p3t)


def _mlp_body(x_ref, w_ref, b_ref, g_ref, bt_ref, o_ref):
    h = jnp.dot(x_ref[...], w_ref[...], preferred_element_type=jnp.float32)
    h = h + b_ref[...]
    mu = jnp.mean(h, axis=-1, keepdims=True)
    var = jnp.mean((h - mu) ** 2, axis=-1, keepdims=True)
    h = (h - mu) / jnp.sqrt(var + 1e-5) * g_ref[...] + bt_ref[...]
    o_ref[...] = jnp.maximum(h, 0.0)


def _mlp(xpad, W, b, gamma, beta):
    blk = 1024
    grid = _NPAD // blk
    return pl.pallas_call(
        _mlp_body,
        grid=(grid,),
        in_specs=[
            pl.BlockSpec((blk, _IN_C), lambda i: (i, 0)),
            pl.BlockSpec((_IN_C, _OUT_C), lambda i: (0, 0)),
            pl.BlockSpec((1, _OUT_C), lambda i: (0, 0)),
            pl.BlockSpec((1, _OUT_C), lambda i: (0, 0)),
            pl.BlockSpec((1, _OUT_C), lambda i: (0, 0)),
        ],
        out_specs=pl.BlockSpec((blk, _OUT_C), lambda i: (i, 0)),
        out_shape=jax.ShapeDtypeStruct((_NPAD, _OUT_C), jnp.float32),
    )(xpad, W, b.reshape(1, -1), gamma.reshape(1, -1), beta.reshape(1, -1))


_NW = 32
_QPW = _MPAD // _NW
_CH = 8
_NCH = _QPW // _CH


def _scgm_body(h_hbm, col_hbm, ids_hbm, batch_hbm, xout_hbm, sb_hbm,
               colv, rows0, rows1, outv, idsv, sbv, sem0, sem1):
    wid = lax.axis_index("s") * 2 + lax.axis_index("c")
    base = wid * _QPW

    pltpu.sync_copy(col_hbm.at[pl.ds(base * _K, _QPW * _K)], colv)
    pltpu.sync_copy(ids_hbm.at[pl.ds(base, _QPW)], idsv)

    pltpu.async_copy(batch_hbm.at[idsv], sbv, sem0).wait()
    pltpu.sync_copy(sbv, sb_hbm.at[pl.ds(base, _QPW)])

    bufs = (rows0, rows1)
    sems = (sem0, sem1)

    def start(t):
        return pltpu.async_copy(
            h_hbm.at[colv.at[pl.ds(t * _CH * _K, _CH * _K)]],
            bufs[t % 2], sems[t % 2])

    cps = {0: start(0)}
    for t in range(_NCH):
        if t + 1 < _NCH:
            cps[t + 1] = start(t + 1)
        cps[t].wait()
        buf = bufs[t % 2]

        def qbody(q, _):
            for c in range(8):
                acc = buf[q * _K, pl.ds(c * 16, 16)]
                for r in range(1, _K):
                    acc = jnp.maximum(acc, buf[q * _K + r, pl.ds(c * 16, 16)])
                outv[t * _CH + q, pl.ds(c * 16, 16)] = acc
            return 0

        lax.fori_loop(0, _CH, qbody, 0)

    pltpu.sync_copy(outv, xout_hbm.at[pl.ds(base, _QPW)])


def _sc_gather_max(h, col_flat, ids_pad, batch):
    mesh = plsc.VectorSubcoreMesh(core_axis_name="c", subcore_axis_name="s")
    f = pl.kernel(
        _scgm_body,
        mesh=mesh,
        out_type=[
            jax.ShapeDtypeStruct((_MPAD, _OUT_C), jnp.float32),
            jax.ShapeDtypeStruct((_MPAD,), jnp.int32),
        ],
        scratch_types=[
            pltpu.VMEM((_QPW * _K,), jnp.int32),
            pltpu.VMEM((_CH * _K, _OUT_C), jnp.float32),
            pltpu.VMEM((_CH * _K, _OUT_C), jnp.float32),
            pltpu.VMEM((_QPW, _OUT_C), jnp.float32),
            pltpu.VMEM((_QPW,), jnp.int32),
            pltpu.VMEM((_QPW,), jnp.int32),
            pltpu.SemaphoreType.DMA,
            pltpu.SemaphoreType.DMA,
        ],
    )
    return f(h, col_flat, ids_pad, batch)


def kernel(x, pos, batch, W, b, gamma, beta):
    ppad = jnp.pad(pos, ((0, _NPAD - _N), (0, 0)))
    px = ppad[:, 0].reshape(_ROWS, 128)
    py = ppad[:, 1].reshape(_ROWS, 128)
    pz = ppad[:, 2].reshape(_ROWS, 128)

    id_clusters, sx, sy, sz = _fps(px, py, pz)
    spos = jnp.stack([sx, sy, sz], axis=1)

    spos_pad = jnp.pad(spos, ((0, _MPAD - _M), (0, 0)))
    col = _knn(spos_pad, ppad.T)

    xpad = jnp.pad(x, ((0, _NPAD - _N), (0, 0)))
    h = _mlp(xpad, W, b, gamma, beta)

    ids_pad = jnp.pad(id_clusters, (0, _MPAD - _M))
    x_out_p, sub_batch_p = _sc_gather_max(h, col.reshape(-1), ids_pad, batch)
    return (x_out_p[:_M], spos, sub_batch_p[:_M])


# speedup vs baseline: 1.3463x; 1.3463x over previous
import functools

import jax
import jax.numpy as jnp
import numpy as np
from jax import lax
from jax.experimental import pallas as pl
from jax.experimental.pallas import tpu as pltpu
from jax.experimental.pallas import tpu_sc as plsc

_N = 10000
_NPAD = 10240
_ROWS = _NPAD // 128
_IN_C = 128
_OUT_C = 128
_M = 2500
_MPAD = 2560
_K = 16
_QB = 256


def _fps_body(px_ref, py_ref, pz_ref, pxs_ref, pys_ref, pzs_ref,
              idx_ref, sx_ref, sy_ref, sz_ref):
    iota = jax.lax.broadcasted_iota(jnp.int32, (_ROWS, 128), 0) * 128 + \
           jax.lax.broadcasted_iota(jnp.int32, (_ROWS, 128), 1)
    iota_f = iota.astype(jnp.float32)
    valid = iota < _N
    BIGF = jnp.float32(1 << 30)

    px0 = px_ref[0, 0]
    py0 = py_ref[0, 0]
    pz0 = pz_ref[0, 0]
    dx = px_ref[...] - px0
    dy = py_ref[...] - py0
    dz = pz_ref[...] - pz0
    d = jnp.where(valid, dx * dx + dy * dy + dz * dz, -jnp.inf)
    idx_ref[0] = 0
    sx_ref[0] = px0
    sy_ref[0] = py0
    sz_ref[0] = pz0

    def body(i, d):
        maxd = jnp.max(d)
        nxtf = jnp.min(jnp.where(d == maxd, iota_f, BIGF))
        nxt = nxtf.astype(jnp.int32)
        pxv = pxs_ref[nxt]
        pyv = pys_ref[nxt]
        pzv = pzs_ref[nxt]
        idx_ref[i] = nxt
        sx_ref[i] = pxv
        sy_ref[i] = pyv
        sz_ref[i] = pzv
        dx = px_ref[...] - pxv
        dy = py_ref[...] - pyv
        dz = pz_ref[...] - pzv
        return jnp.minimum(d, dx * dx + dy * dy + dz * dz)

    jax.lax.fori_loop(1, _M, body, d)


def _fps(px, py, pz, pxs, pys, pzs):
    return pl.pallas_call(
        _fps_body,
        in_specs=[pl.BlockSpec(memory_space=pltpu.VMEM)] * 3
        + [pl.BlockSpec(memory_space=pltpu.SMEM)] * 3,
        out_specs=[pl.BlockSpec(memory_space=pltpu.SMEM)] * 4,
        out_shape=[
            jax.ShapeDtypeStruct((_M,), jnp.int32),
            jax.ShapeDtypeStruct((_M,), jnp.float32),
            jax.ShapeDtypeStruct((_M,), jnp.float32),
            jax.ShapeDtypeStruct((_M,), jnp.float32),
        ],
    )(px, py, pz, pxs, pys, pzs)


def _knn_body(q_ref, p_ref, out_ref, d2_ref):
    col_iota = jax.lax.broadcasted_iota(jnp.int32, (_QB, _NPAD), 1)
    lane16 = jax.lax.broadcasted_iota(jnp.int32, (_QB, _K), 1)
    BIG = jnp.int32(1 << 30)
    INF = jnp.float32(jnp.inf)

    ptx = p_ref[0:1, :]
    pty = p_ref[1:2, :]
    ptz = p_ref[2:3, :]
    pnorm = ptx * ptx + pty * pty + ptz * ptz
    pn_iota = jax.lax.broadcasted_iota(jnp.int32, (1, _NPAD), 1)
    pnorm = jnp.where(pn_iota < _N, pnorm, INF)

    q = q_ref[...]
    qx = q[:, 0:1]
    qy = q[:, 1:2]
    qz = q[:, 2:3]
    qnorm = qx * qx + qy * qy + qz * qz

    cross = jnp.dot(q, p_ref[...], preferred_element_type=jnp.float32)
    d2_ref[...] = (qnorm + pnorm) - 2.0 * cross

    def body(k, carry):
        acc, prev = carry
        d2 = d2_ref[...]
        d2 = jnp.where(col_iota == prev, INF, d2)
        d2_ref[...] = d2
        mn = jnp.min(d2, axis=1, keepdims=True)
        idx = jnp.min(jnp.where(d2 == mn, col_iota, BIG), axis=1, keepdims=True)
        acc = jnp.where(lane16 == k, idx, acc)
        return acc, idx

    acc0 = jnp.zeros((_QB, _K), jnp.int32)
    prev0 = jnp.full((_QB, 1), -1, jnp.int32)
    acc, _ = jax.lax.fori_loop(0, _K, body, (acc0, prev0))
    out_ref[...] = acc


def _knn(spos_pad, p3t):
    return pl.pallas_call(
        _knn_body,
        grid=(_MPAD // _QB,),
        in_specs=[
            pl.BlockSpec((_QB, 3), lambda i: (i, 0)),
            pl.BlockSpec((3, _NPAD), lambda i: (0, 0)),
        ],
        out_specs=pl.BlockSpec((_QB, _K), lambda i: (i, 0)),
        out_shape=jax.ShapeDtypeStruct((_MPAD, _K), jnp.int32),
        scratch_shapes=[pltpu.VMEM((_QB, _NPAD), jnp.float32)],
    )(spos_pad, p3t)


def _mlp_body(x_ref, w_ref, b_ref, g_ref, bt_ref, o_ref):
    h = jnp.dot(x_ref[...], w_ref[...], preferred_element_type=jnp.float32)
    h = h + b_ref[...]
    mu = jnp.mean(h, axis=-1, keepdims=True)
    var = jnp.mean((h - mu) ** 2, axis=-1, keepdims=True)
    h = (h - mu) / jnp.sqrt(var + 1e-5) * g_ref[...] + bt_ref[...]
    o_ref[...] = jnp.maximum(h, 0.0)


def _mlp(xpad, W, b, gamma, beta):
    blk = 1024
    grid = _NPAD // blk
    return pl.pallas_call(
        _mlp_body,
        grid=(grid,),
        in_specs=[
            pl.BlockSpec((blk, _IN_C), lambda i: (i, 0)),
            pl.BlockSpec((_IN_C, _OUT_C), lambda i: (0, 0)),
            pl.BlockSpec((1, _OUT_C), lambda i: (0, 0)),
            pl.BlockSpec((1, _OUT_C), lambda i: (0, 0)),
            pl.BlockSpec((1, _OUT_C), lambda i: (0, 0)),
        ],
        out_specs=pl.BlockSpec((blk, _OUT_C), lambda i: (i, 0)),
        out_shape=jax.ShapeDtypeStruct((_NPAD, _OUT_C), jnp.float32),
    )(xpad, W, b.reshape(1, -1), gamma.reshape(1, -1), beta.reshape(1, -1))


_NW = 32
_QPW = _MPAD // _NW
_CH = 8
_NCH = _QPW // _CH


def _scgm_body(h_hbm, col_hbm, ids_hbm, batch_hbm, xout_hbm, sb_hbm,
               colv, rows0, rows1, outv, idsv, sbv, sem0, sem1):
    wid = lax.axis_index("s") * 2 + lax.axis_index("c")
    base = wid * _QPW

    pltpu.sync_copy(col_hbm.at[pl.ds(base * _K, _QPW * _K)], colv)
    pltpu.sync_copy(ids_hbm.at[pl.ds(base, _QPW)], idsv)

    pltpu.async_copy(batch_hbm.at[idsv], sbv, sem0).wait()
    pltpu.sync_copy(sbv, sb_hbm.at[pl.ds(base, _QPW)])

    bufs = (rows0, rows1)
    sems = (sem0, sem1)

    def start(t):
        return pltpu.async_copy(
            h_hbm.at[colv.at[pl.ds(t * _CH * _K, _CH * _K)]],
            bufs[t % 2], sems[t % 2])

    cps = {0: start(0)}
    for t in range(_NCH):
        if t + 1 < _NCH:
            cps[t + 1] = start(t + 1)
        cps[t].wait()
        buf = bufs[t % 2]

        def qbody(q, _):
            for c in range(8):
                acc = buf[q * _K, pl.ds(c * 16, 16)]
                for r in range(1, _K):
                    acc = jnp.maximum(acc, buf[q * _K + r, pl.ds(c * 16, 16)])
                outv[t * _CH + q, pl.ds(c * 16, 16)] = acc
            return 0

        lax.fori_loop(0, _CH, qbody, 0)

    pltpu.sync_copy(outv, xout_hbm.at[pl.ds(base, _QPW)])


def _sc_gather_max(h, col_flat, ids_pad, batch):
    mesh = plsc.VectorSubcoreMesh(core_axis_name="c", subcore_axis_name="s")
    f = pl.kernel(
        _scgm_body,
        mesh=mesh,
        out_type=[
            jax.ShapeDtypeStruct((_MPAD, _OUT_C), jnp.float32),
            jax.ShapeDtypeStruct((_MPAD,), jnp.int32),
        ],
        scratch_types=[
            pltpu.VMEM((_QPW * _K,), jnp.int32),
            pltpu.VMEM((_CH * _K, _OUT_C), jnp.float32),
            pltpu.VMEM((_CH * _K, _OUT_C), jnp.float32),
            pltpu.VMEM((_QPW, _OUT_C), jnp.float32),
            pltpu.VMEM((_QPW,), jnp.int32),
            pltpu.VMEM((_QPW,), jnp.int32),
            pltpu.SemaphoreType.DMA,
            pltpu.SemaphoreType.DMA,
        ],
    )
    return f(h, col_flat, ids_pad, batch)


def kernel(x, pos, batch, W, b, gamma, beta):
    ppad = jnp.pad(pos, ((0, _NPAD - _N), (0, 0)))
    px = ppad[:, 0].reshape(_ROWS, 128)
    py = ppad[:, 1].reshape(_ROWS, 128)
    pz = ppad[:, 2].reshape(_ROWS, 128)

    id_clusters, sx, sy, sz = _fps(px, py, pz, ppad[:, 0], ppad[:, 1],
                                   ppad[:, 2])
    spos = jnp.stack([sx, sy, sz], axis=1)

    spos_pad = jnp.pad(spos, ((0, _MPAD - _M), (0, 0)))
    col = _knn(spos_pad, ppad.T)

    xpad = jnp.pad(x, ((0, _NPAD - _N), (0, 0)))
    h = _mlp(xpad, W, b, gamma, beta)

    ids_pad = jnp.pad(id_clusters, (0, _MPAD - _M))
    x_out_p, sub_batch_p = _sc_gather_max(h, col.reshape(-1), ids_pad, batch)
    return (x_out_p[:_M], spos, sub_batch_p[:_M])
